# baseline (speedup 1.0000x reference)
import jax
import jax.numpy as jnp
from jax.experimental import pallas as pl
from jax.experimental.pallas import tpu as pltpu

_N = 10000
_D = 128
_H = 32
_OUT = 128
_BK = 200
_STEPS = _N // _BK


def _outer(x_hbm, adj_hbm, w1_ref, b1_ref, gamma_ref, beta_ref,
           w2_ref, b2_ref, wfc_ref, bfc_ref, out_ref,
           y_ref, zt_ref, cnt_ref):
    cnt_ref[0] = 0
    zt_ref[...] = jnp.zeros_like(zt_ref)

    def body(x_blk, adj_blk):
        k = cnt_ref[0]
        xb = x_blk[...]
        adjb = adj_blk[...]
        yb = jax.lax.dot_general(
            xb, w1_ref[...], (((1,), (1,)), ((), ())),
            preferred_element_type=jnp.float32,
            precision=jax.lax.Precision.DEFAULT)
        y_ref[pl.ds(k * _BK, _BK), :] = yb
        zt_ref[...] += jax.lax.dot_general(
            yb, adjb, (((0,), (0,)), ((), ())),
            preferred_element_type=jnp.float32,
            precision=jax.lax.Precision.DEFAULT)
        cnt_ref[0] = k + 1

    pltpu.emit_pipeline(
        body,
        grid=(_STEPS,),
        in_specs=[
            pl.BlockSpec((_BK, _D), lambda k: (k, 0)),
            pl.BlockSpec((_BK, _N), lambda k: (k, 0),
                         pipeline_mode=pl.Buffered(buffer_count=6)),
        ],
    )(x_hbm, adj_hbm)

    b1c = b1_ref[...][:, None]
    gammac = gamma_ref[...][:, None]
    betac = beta_ref[...][:, None]
    b2c = b2_ref[...][:, None]
    yt = y_ref[...].T
    ht = yt + zt_ref[...] + b1c
    mu = jnp.mean(ht, axis=1, keepdims=True)
    d = ht - mu
    var = jnp.mean(d * d, axis=1, keepdims=True)
    hn = d * jax.lax.rsqrt(var + 1e-5) * gammac + betac
    hr = jnp.maximum(hn, 0.0)
    h2 = jax.lax.dot_general(
        w2_ref[...], hr, (((1,), (0,)), ((), ())),
        preferred_element_type=jnp.float32,
        precision=jax.lax.Precision.HIGHEST) + b2c
    h2 = jnp.maximum(h2, 0.0)
    pooled = jnp.mean(h2, axis=1, keepdims=True)
    out = jax.lax.dot_general(
        pooled, wfc_ref[...], (((0,), (1,)), ((), ())),
        preferred_element_type=jnp.float32,
        precision=jax.lax.Precision.HIGHEST) + bfc_ref[...][None, :]
    out_ref[...] = out


def kernel(x, adj, W1, b1, gamma, beta, W2, b2, Wfc, bfc):
    return pl.pallas_call(
        _outer,
        in_specs=[
            pl.BlockSpec(memory_space=pl.ANY),
            pl.BlockSpec(memory_space=pl.ANY),
            pl.BlockSpec((_H, _D), lambda: (0, 0)),
            pl.BlockSpec((_H,), lambda: (0,)),
            pl.BlockSpec((_H,), lambda: (0,)),
            pl.BlockSpec((_H,), lambda: (0,)),
            pl.BlockSpec((_H, _H), lambda: (0, 0)),
            pl.BlockSpec((_H,), lambda: (0,)),
            pl.BlockSpec((_OUT, _H), lambda: (0, 0)),
            pl.BlockSpec((_OUT,), lambda: (0,)),
        ],
        out_specs=pl.BlockSpec((1, _OUT), lambda: (0, 0)),
        out_shape=jax.ShapeDtypeStruct((1, _OUT), jnp.float32),
        scratch_shapes=[
            pltpu.VMEM((_N, _H), jnp.float32),
            pltpu.VMEM((_H, _N), jnp.float32),
            pltpu.SMEM((1,), jnp.int32),
        ],
        compiler_params=pltpu.CompilerParams(
            dimension_semantics=(),
            vmem_limit_bytes=64 * 1024 * 1024),
    )(x, adj, W1, b1, gamma, beta, W2, b2, Wfc, bfc)


# speedup vs baseline: 1.0264x; 1.0264x over previous
import jax
import jax.numpy as jnp
from jax.experimental import pallas as pl
from jax.experimental.pallas import tpu as pltpu

_N = 10000
_D = 128
_H = 32
_OUT = 128
_BK = 400
_STEPS = _N // _BK


def _gnn_kernel(x_ref, adj_ref, w1_ref, b1_ref, gamma_ref, beta_ref,
                w2_ref, b2_ref, wfc_ref, bfc_ref, out_ref,
                y_ref, zt_ref):
    k = pl.program_id(0)

    xb = x_ref[...]
    adjb = adj_ref[...]

    yb = jax.lax.dot_general(
        xb, w1_ref[...], (((1,), (1,)), ((), ())),
        preferred_element_type=jnp.float32,
        precision=jax.lax.Precision.DEFAULT)
    y_ref[pl.ds(k * _BK, _BK), :] = yb

    zpart = jax.lax.dot_general(
        yb, adjb,
        (((0,), (0,)), ((), ())),
        preferred_element_type=jnp.float32,
        precision=jax.lax.Precision.DEFAULT)

    @pl.when(k == 0)
    def _():
        zt_ref[...] = zpart

    @pl.when(k > 0)
    def _():
        zt_ref[...] += zpart

    @pl.when(k == _STEPS - 1)
    def _():
        b1c = b1_ref[...][:, None]
        gammac = gamma_ref[...][:, None]
        betac = beta_ref[...][:, None]
        b2c = b2_ref[...][:, None]
        yt = y_ref[...].T
        ht = yt + zt_ref[...] + b1c
        mu = jnp.mean(ht, axis=1, keepdims=True)
        d = ht - mu
        var = jnp.mean(d * d, axis=1, keepdims=True)
        hn = d * jax.lax.rsqrt(var + 1e-5) * gammac + betac
        hr = jnp.maximum(hn, 0.0)
        h2 = jax.lax.dot_general(
            w2_ref[...], hr, (((1,), (0,)), ((), ())),
            preferred_element_type=jnp.float32,
            precision=jax.lax.Precision.HIGHEST) + b2c
        h2 = jnp.maximum(h2, 0.0)
        pooled = jnp.mean(h2, axis=1, keepdims=True)
        out = jax.lax.dot_general(
            pooled, wfc_ref[...], (((0,), (1,)), ((), ())),
            preferred_element_type=jnp.float32,
            precision=jax.lax.Precision.HIGHEST) + bfc_ref[...][None, :]
        out_ref[...] = out


def kernel(x, adj, W1, b1, gamma, beta, W2, b2, Wfc, bfc):
    return pl.pallas_call(
        _gnn_kernel,
        grid=(_STEPS,),
        in_specs=[
            pl.BlockSpec((_BK, _D), lambda k: (k, 0)),
            pl.BlockSpec((_BK, _N), lambda k: (k, 0)),
            pl.BlockSpec((_H, _D), lambda k: (0, 0)),
            pl.BlockSpec((_H,), lambda k: (0,)),
            pl.BlockSpec((_H,), lambda k: (0,)),
            pl.BlockSpec((_H,), lambda k: (0,)),
            pl.BlockSpec((_H, _H), lambda k: (0, 0)),
            pl.BlockSpec((_H,), lambda k: (0,)),
            pl.BlockSpec((_OUT, _H), lambda k: (0, 0)),
            pl.BlockSpec((_OUT,), lambda k: (0,)),
        ],
        out_specs=pl.BlockSpec((1, _OUT), lambda k: (0, 0)),
        out_shape=jax.ShapeDtypeStruct((1, _OUT), jnp.float32),
        scratch_shapes=[
            pltpu.VMEM((_N, _H), jnp.float32),
            pltpu.VMEM((_H, _N), jnp.float32),
        ],
        compiler_params=pltpu.CompilerParams(
            dimension_semantics=("arbitrary",)),
    )(x, adj, W1, b1, gamma, beta, W2, b2, Wfc, bfc)


# speedup vs baseline: 1.0307x; 1.0041x over previous
import jax
import jax.numpy as jnp
from jax.experimental import pallas as pl
from jax.experimental.pallas import tpu as pltpu

_N = 10000
_D = 128
_H = 32
_OUT = 128
_BK = 400
_STEPS = _N // _BK


def _gnn_kernel(x_ref, adj_ref, w1_ref, b1_ref, gamma_ref, beta_ref,
                w2_ref, b2_ref, wfc_ref, bfc_ref, out_ref, zt_ref):
    k = pl.program_id(0)

    xb = x_ref[pl.ds(k * _BK, _BK), :]
    adjb = adj_ref[...]

    yb = jax.lax.dot_general(
        xb, w1_ref[...], (((1,), (1,)), ((), ())),
        preferred_element_type=jnp.float32,
        precision=jax.lax.Precision.DEFAULT)

    zpart = jax.lax.dot_general(
        yb, adjb,
        (((0,), (0,)), ((), ())),
        preferred_element_type=jnp.float32,
        precision=jax.lax.Precision.DEFAULT)

    @pl.when(k == 0)
    def _():
        zt_ref[...] = zpart

    @pl.when(k > 0)
    def _():
        zt_ref[...] += zpart

    @pl.when(k == _STEPS - 1)
    def _():
        b1c = b1_ref[...][:, None]
        gammac = gamma_ref[...][:, None]
        betac = beta_ref[...][:, None]
        b2c = b2_ref[...][:, None]
        yt = jax.lax.dot_general(
            w1_ref[...], x_ref[...], (((1,), (1,)), ((), ())),
            preferred_element_type=jnp.float32,
            precision=jax.lax.Precision.DEFAULT)
        ht = yt + zt_ref[...] + b1c
        mu = jnp.mean(ht, axis=1, keepdims=True)
        d = ht - mu
        var = jnp.mean(d * d, axis=1, keepdims=True)
        hn = d * jax.lax.rsqrt(var + 1e-5) * gammac + betac
        hr = jnp.maximum(hn, 0.0)
        h2 = jax.lax.dot_general(
            w2_ref[...], hr, (((1,), (0,)), ((), ())),
            preferred_element_type=jnp.float32,
            precision=jax.lax.Precision.HIGHEST) + b2c
        h2 = jnp.maximum(h2, 0.0)
        pooled = jnp.mean(h2, axis=1, keepdims=True)
        out = jax.lax.dot_general(
            pooled, wfc_ref[...], (((0,), (1,)), ((), ())),
            preferred_element_type=jnp.float32,
            precision=jax.lax.Precision.HIGHEST) + bfc_ref[...][None, :]
        out_ref[...] = out


def kernel(x, adj, W1, b1, gamma, beta, W2, b2, Wfc, bfc):
    return pl.pallas_call(
        _gnn_kernel,
        grid=(_STEPS,),
        in_specs=[
            pl.BlockSpec((_N, _D), lambda k: (0, 0)),
            pl.BlockSpec((_BK, _N), lambda k: (k, 0)),
            pl.BlockSpec((_H, _D), lambda k: (0, 0)),
            pl.BlockSpec((_H,), lambda k: (0,)),
            pl.BlockSpec((_H,), lambda k: (0,)),
            pl.BlockSpec((_H,), lambda k: (0,)),
            pl.BlockSpec((_H, _H), lambda k: (0, 0)),
            pl.BlockSpec((_H,), lambda k: (0,)),
            pl.BlockSpec((_OUT, _H), lambda k: (0, 0)),
            pl.BlockSpec((_OUT,), lambda k: (0,)),
        ],
        out_specs=pl.BlockSpec((1, _OUT), lambda k: (0, 0)),
        out_shape=jax.ShapeDtypeStruct((1, _OUT), jnp.float32),
        scratch_shapes=[
            pltpu.VMEM((_H, _N), jnp.float32),
        ],
        compiler_params=pltpu.CompilerParams(
            dimension_semantics=("arbitrary",)),
    )(x, adj, W1, b1, gamma, beta, W2, b2, Wfc, bfc)
